# Initial kernel scaffold; baseline (speedup 1.0000x reference)
import jax
import jax.numpy as jnp
from jax.experimental import pallas as pl


def kernel(x, edge_index, W_l0, W_r0, b0, W_l1, W_r1, b1, W_l2, W_r2, b2):
    raise NotImplementedError("write your pallas kernel here")


# speedup vs baseline: 4.2762x; 4.2762x over previous
import jax
import jax.numpy as jnp
from jax import lax
from jax.experimental import pallas as pl
from jax.experimental.pallas import tpu as pltpu
from jax.experimental.pallas import tpu_sc as plsc

_N = 10000
_D = 128
_H = _D // 2
_E = 320000
_NP = 10240
_NC = 2
_NS = 16
_KC = 128
_CHT = _E // _KC
_RPT = _NP // _NS


def _make_sc_agg(with_deg: bool):
    out_type = [jax.ShapeDtypeStruct((_NC, _NP, _H), jnp.float32)]
    scratch = [
        pltpu.VMEM((_KC,), jnp.int32),
        pltpu.VMEM((_KC,), jnp.int32),
        pltpu.VMEM((_KC, _H), jnp.float32),
        pltpu.SemaphoreType.DMA,
        pltpu.VMEM_SHARED((_NP, _H), jnp.float32),
    ]
    if with_deg:
        out_type.append(jax.ShapeDtypeStruct((_NC, _NP), jnp.float32))
        scratch.append(pltpu.VMEM_SHARED((_NP,), jnp.float32))
        scratch.append(pltpu.VMEM((_KC,), jnp.float32))
        scratch.append(pltpu.VMEM((_RPT,), jnp.float32))
    mesh = plsc.VectorSubcoreMesh(core_axis_name="c", subcore_axis_name="s")

    def body(x0_hbm, x1_hbm, ei_hbm, out_hbm, *rest):
        if with_deg:
            deg_hbm, srcv, dstv, rows, sem, agg_sh, deg_sh, onesv, zcol = rest
        else:
            srcv, dstv, rows, sem, agg_sh = rest
        cid = lax.axis_index("c")
        sid = lax.axis_index("s")
        zero16 = jnp.zeros((16,), jnp.float32)
        ones16 = jnp.ones((16,), jnp.float32)

        def zrow(i, carry):
            rows[i // (_H // 16), pl.ds((i % (_H // 16)) * 16, 16)] = zero16
            return carry
        lax.fori_loop(0, _KC * (_H // 16), zrow, 0)
        base = sid * _RPT
        for r in range(_RPT // _KC):
            pltpu.sync_copy(rows, agg_sh.at[pl.ds(base + r * _KC, _KC)])
        if with_deg:
            def zdeg(i, carry):
                zcol[pl.ds(i * 16, 16)] = zero16
                return carry
            lax.fori_loop(0, _RPT // 16, zdeg, 0)

            def fones(i, carry):
                onesv[pl.ds(i * 16, 16)] = ones16
                return carry
            lax.fori_loop(0, _KC // 16, fones, 0)
            pltpu.sync_copy(zcol, deg_sh.at[pl.ds(base, _RPT)])
        plsc.subcore_barrier()

        cbase = 156 * sid + jnp.minimum(sid, 4)
        nch = 156 + jnp.where(sid < 4, 1, 0)

        def run(xtab, do_deg):
            def chunk(c, carry):
                cb = (cbase + c) * _KC
                pltpu.sync_copy(ei_hbm.at[pl.ds(cb, _KC)], srcv)
                pltpu.sync_copy(ei_hbm.at[pl.ds(_E + cb, _KC)], dstv)
                pltpu.async_copy(xtab.at[srcv], rows, sem).wait()
                pltpu.sync_copy(rows, agg_sh.at[dstv], add=True)
                if do_deg:
                    pltpu.sync_copy(onesv, deg_sh.at[dstv], add=True)
                return carry
            lax.fori_loop(0, nch, chunk, 0)

        pl.when(cid == 0)(lambda: run(x0_hbm, with_deg))
        pl.when(cid == 1)(lambda: run(x1_hbm, False))

        plsc.subcore_barrier()
        pltpu.sync_copy(agg_sh.at[pl.ds(base, _RPT)],
                        out_hbm.at[cid, pl.ds(base, _RPT), :])
        if with_deg:
            pltpu.sync_copy(deg_sh.at[pl.ds(base, _RPT)],
                            deg_hbm.at[cid, pl.ds(base, _RPT)])

    return pl.kernel(body, out_type=out_type, mesh=mesh,
                     scratch_types=scratch,
                     compiler_params=pltpu.CompilerParams(
                         use_tc_tiling_on_sc=False))


_SC_AGG_DEG = _make_sc_agg(True)
_SC_AGG = _make_sc_agg(False)

_BN = 1000


def _make_dense(relu: bool, split_out: bool):
    def body(p0, p1, dt, x0r, x1r, wlr, wrr, br, *outs):
        deg = jnp.sum(dt[...], axis=1, keepdims=True)
        inv = 1.0 / jnp.maximum(deg, 1.0)
        agg = jnp.concatenate([p0[0, :, :], p1[0, :, :]], axis=1) * inv
        xr = jnp.concatenate([x0r[...], x1r[...]], axis=1)
        h = (jnp.dot(agg, wlr[...], preferred_element_type=jnp.float32,
                     precision=lax.Precision.HIGHEST)
             + jnp.dot(xr, wrr[...], preferred_element_type=jnp.float32,
                       precision=lax.Precision.HIGHEST)
             + br[...])
        if relu:
            h = jnp.maximum(h, 0.0)
        h = h + xr
        if split_out:
            outs[0][...] = h[:, :_H]
            outs[1][...] = h[:, _H:]
        else:
            outs[0][...] = h

    if split_out:
        out_shape = [jax.ShapeDtypeStruct((_N, _H), jnp.float32)] * 2
        out_specs = [pl.BlockSpec((_BN, _H), lambda i: (i, 0))] * 2
    else:
        out_shape = jax.ShapeDtypeStruct((_N, _D), jnp.float32)
        out_specs = pl.BlockSpec((_BN, _D), lambda i: (i, 0))
    return pl.pallas_call(
        body,
        grid=(_N // _BN,),
        in_specs=[
            pl.BlockSpec((1, _BN, _H), lambda i: (0, i, 0)),
            pl.BlockSpec((1, _BN, _H), lambda i: (1, i, 0)),
            pl.BlockSpec((_BN, _NC), lambda i: (i, 0)),
            pl.BlockSpec((_BN, _H), lambda i: (i, 0)),
            pl.BlockSpec((_BN, _H), lambda i: (i, 0)),
            pl.BlockSpec((_D, _D), lambda i: (0, 0)),
            pl.BlockSpec((_D, _D), lambda i: (0, 0)),
            pl.BlockSpec((1, _D), lambda i: (0, 0)),
        ],
        out_specs=out_specs,
        out_shape=out_shape,
    )


_DENSE_RELU = _make_dense(True, True)
_DENSE_LAST = _make_dense(False, False)


def kernel(x, edge_index, W_l0, W_r0, b0, W_l1, W_r1, b1, W_l2, W_r2, b2):
    ei = edge_index.reshape(2 * _E)
    x0 = x[:, :_H]
    x1 = x[:, _H:]
    agg, degp = _SC_AGG_DEG(x0, x1, ei)
    degp_t = degp.T
    h0, h1 = _DENSE_RELU(agg, agg, degp_t, x0, x1, W_l0, W_r0,
                         b0.reshape(1, _D))
    agg, = _SC_AGG(h0, h1, ei)
    h0, h1 = _DENSE_RELU(agg, agg, degp_t, h0, h1, W_l1, W_r1,
                         b1.reshape(1, _D))
    agg, = _SC_AGG(h0, h1, ei)
    h = _DENSE_LAST(agg, agg, degp_t, h0, h1, W_l2, W_r2,
                    b2.reshape(1, _D))
    return h


# speedup vs baseline: 5.2140x; 1.2193x over previous
import jax
import jax.numpy as jnp
from jax import lax
from jax.experimental import pallas as pl
from jax.experimental.pallas import tpu as pltpu
from jax.experimental.pallas import tpu_sc as plsc

_N = 10000
_D = 128
_H = _D // 2
_E = 320000
_NP = 10240
_NC = 2
_NS = 16
_KC = 128
_CPT = 160
_CH = _NS * _CPT
_EP = _CH * _KC
_NB = 4
_RPT = _NP // _NS


def _make_sc_agg(with_deg: bool):
    out_type = [jax.ShapeDtypeStruct((_NC, _NP, _H), jnp.float32)]
    scratch = [
        pltpu.VMEM((_CPT, _KC), jnp.int32),
        pltpu.VMEM((_CPT, _KC), jnp.int32),
        [pltpu.VMEM((_KC, _H), jnp.float32)] * _NB,
        [pltpu.SemaphoreType.DMA] * _NB,
        pltpu.SemaphoreType.DMA,
        pltpu.VMEM_SHARED((_NP, _H), jnp.float32),
    ]
    if with_deg:
        out_type.append(jax.ShapeDtypeStruct((_NC, _NP), jnp.float32))
        scratch.append(pltpu.VMEM_SHARED((_NP,), jnp.float32))
        scratch.append(pltpu.VMEM((_KC,), jnp.float32))
        scratch.append(pltpu.VMEM((_RPT,), jnp.float32))
    mesh = plsc.VectorSubcoreMesh(core_axis_name="c", subcore_axis_name="s")

    def body(x0_hbm, x1_hbm, src_hbm, dst_hbm, out_hbm, *rest):
        if with_deg:
            (deg_hbm, srcl, dstl, ring, gsems, isem, agg_sh,
             deg_sh, onesv, zcol) = rest
        else:
            srcl, dstl, ring, gsems, isem, agg_sh = rest
        cid = lax.axis_index("c")
        sid = lax.axis_index("s")
        zero16 = jnp.zeros((16,), jnp.float32)
        ones16 = jnp.ones((16,), jnp.float32)

        crow = sid * _CPT
        pltpu.async_copy(src_hbm.at[pl.ds(crow, _CPT), :], srcl, isem)
        pltpu.async_copy(dst_hbm.at[pl.ds(crow, _CPT), :], dstl, isem)

        def zrow(i, carry):
            ring[0][i // (_H // 16), pl.ds((i % (_H // 16)) * 16, 16)] = zero16
            return carry
        lax.fori_loop(0, _KC * (_H // 16), zrow, 0)
        base = sid * _RPT
        for r in range(_RPT // _KC):
            pltpu.sync_copy(ring[0], agg_sh.at[pl.ds(base + r * _KC, _KC)])
        if with_deg:
            def zdeg(i, carry):
                zcol[pl.ds(i * 16, 16)] = zero16
                return carry
            lax.fori_loop(0, _RPT // 16, zdeg, 0)

            def fones(i, carry):
                onesv[pl.ds(i * 16, 16)] = ones16
                return carry
            lax.fori_loop(0, _KC // 16, fones, 0)
            pltpu.sync_copy(zcol, deg_sh.at[pl.ds(base, _RPT)])
        pltpu.make_async_copy(src_hbm.at[pl.ds(crow, _CPT), :], srcl,
                              isem).wait()
        pltpu.make_async_copy(dst_hbm.at[pl.ds(crow, _CPT), :], dstl,
                              isem).wait()
        plsc.subcore_barrier()

        def run(xtab, do_deg):
            for b in range(_NB):
                pltpu.async_copy(xtab.at[srcl.at[b]], ring[b], gsems[b])

            def step(i, carry):
                for b in range(_NB):
                    cc = i * _NB + b
                    pltpu.make_async_copy(xtab.at[srcl.at[0]], ring[b],
                                          gsems[b]).wait()
                    pltpu.sync_copy(ring[b], agg_sh.at[dstl.at[cc]],
                                    add=True)
                    if do_deg:
                        pltpu.sync_copy(onesv, deg_sh.at[dstl.at[cc]],
                                        add=True)
                    nc = cc + _NB

                    @pl.when(nc < _CPT)
                    def _():
                        pltpu.async_copy(xtab.at[srcl.at[nc]], ring[b],
                                         gsems[b])
                return carry
            lax.fori_loop(0, _CPT // _NB, step, 0)

        pl.when(cid == 0)(lambda: run(x0_hbm, with_deg))
        pl.when(cid == 1)(lambda: run(x1_hbm, False))

        plsc.subcore_barrier()
        pltpu.sync_copy(agg_sh.at[pl.ds(base, _RPT)],
                        out_hbm.at[cid, pl.ds(base, _RPT), :])
        if with_deg:
            pltpu.sync_copy(deg_sh.at[pl.ds(base, _RPT)],
                            deg_hbm.at[cid, pl.ds(base, _RPT)])

    return pl.kernel(body, out_type=out_type, mesh=mesh,
                     scratch_types=scratch,
                     compiler_params=pltpu.CompilerParams(
                         use_tc_tiling_on_sc=False))


_SC_AGG_DEG = _make_sc_agg(True)
_SC_AGG = _make_sc_agg(False)

_BN = 1000


def _make_dense(relu: bool, split_out: bool):
    def body(p0, p1, dt, x0r, x1r, wlr, wrr, br, *outs):
        deg = jnp.sum(dt[...], axis=1, keepdims=True)
        inv = 1.0 / jnp.maximum(deg, 1.0)
        agg = jnp.concatenate([p0[0, :, :], p1[0, :, :]], axis=1) * inv
        xr = jnp.concatenate([x0r[...], x1r[...]], axis=1)
        h = (jnp.dot(agg, wlr[...], preferred_element_type=jnp.float32,
                     precision=lax.Precision.HIGHEST)
             + jnp.dot(xr, wrr[...], preferred_element_type=jnp.float32,
                       precision=lax.Precision.HIGHEST)
             + br[...])
        if relu:
            h = jnp.maximum(h, 0.0)
        h = h + xr
        if split_out:
            outs[0][...] = h[:, :_H]
            outs[1][...] = h[:, _H:]
        else:
            outs[0][...] = h

    if split_out:
        out_shape = [jax.ShapeDtypeStruct((_N, _H), jnp.float32)] * 2
        out_specs = [pl.BlockSpec((_BN, _H), lambda i: (i, 0))] * 2
    else:
        out_shape = jax.ShapeDtypeStruct((_N, _D), jnp.float32)
        out_specs = pl.BlockSpec((_BN, _D), lambda i: (i, 0))
    return pl.pallas_call(
        body,
        grid=(_N // _BN,),
        in_specs=[
            pl.BlockSpec((1, _BN, _H), lambda i: (0, i, 0)),
            pl.BlockSpec((1, _BN, _H), lambda i: (1, i, 0)),
            pl.BlockSpec((_BN, _NC), lambda i: (i, 0)),
            pl.BlockSpec((_BN, _H), lambda i: (i, 0)),
            pl.BlockSpec((_BN, _H), lambda i: (i, 0)),
            pl.BlockSpec((_D, _D), lambda i: (0, 0)),
            pl.BlockSpec((_D, _D), lambda i: (0, 0)),
            pl.BlockSpec((1, _D), lambda i: (0, 0)),
        ],
        out_specs=out_specs,
        out_shape=out_shape,
    )


_DENSE_RELU = _make_dense(True, True)
_DENSE_LAST = _make_dense(False, False)


def kernel(x, edge_index, W_l0, W_r0, b0, W_l1, W_r1, b1, W_l2, W_r2, b2):
    npad = _EP - _E
    src2d = jnp.concatenate(
        [edge_index[0], jnp.zeros((npad,), jnp.int32)]).reshape(_CH, _KC)
    dst2d = jnp.concatenate(
        [edge_index[1], jnp.full((npad,), _N, jnp.int32)]).reshape(_CH, _KC)
    x0 = x[:, :_H]
    x1 = x[:, _H:]
    agg, degp = _SC_AGG_DEG(x0, x1, src2d, dst2d)
    degp_t = degp.T
    h0, h1 = _DENSE_RELU(agg, agg, degp_t, x0, x1, W_l0, W_r0,
                         b0.reshape(1, _D))
    agg, = _SC_AGG(h0, h1, src2d, dst2d)
    h0, h1 = _DENSE_RELU(agg, agg, degp_t, h0, h1, W_l1, W_r1,
                         b1.reshape(1, _D))
    agg, = _SC_AGG(h0, h1, src2d, dst2d)
    h = _DENSE_LAST(agg, agg, degp_t, h0, h1, W_l2, W_r2,
                    b2.reshape(1, _D))
    return h
